# baseline (speedup 1.0000x reference)
import functools

import jax
import jax.numpy as jnp
from jax import lax
from jax.experimental import pallas as pl
from jax.experimental.pallas import tpu as pltpu
from jax.experimental.pallas import tpu_sc as plsc

NC = 2
NS = 16
CHUNK = 128
DEGW = 8
HB = 2


def _make_agg(n_pad, d, kb, rem):
  mesh = plsc.VectorSubcoreMesh(core_axis_name="c", subcore_axis_name="s")
  rows_per_tile = n_pad // NS

  @functools.partial(
      pl.kernel,
      mesh=mesh,
      out_type=jax.ShapeDtypeStruct((NC, n_pad, d), jnp.float32),
      compiler_params=pltpu.CompilerParams(use_tc_tiling_on_sc=False),
      scratch_types=[
          pltpu.VMEM((kb * CHUNK,), jnp.int32),
          pltpu.VMEM((kb, CHUNK), jnp.int32),
          pltpu.VMEM((CHUNK,), jnp.int32),
          pltpu.VMEM((CHUNK,), jnp.int32),
          pltpu.VMEM((CHUNK, d), jnp.float32),
          [pltpu.VMEM((CHUNK, d), jnp.float32) for _ in range(HB)],
          pltpu.VMEM_SHARED((n_pad, d), jnp.float32),
          pltpu.VMEM_SHARED((n_pad, d), jnp.float32),
          pltpu.SemaphoreType.DMA,
          [pltpu.SemaphoreType.DMA for _ in range(HB)],
      ],
  )
  def agg(hs_hbm, ei_hbm, dst_hbm, zeros_hbm, out_hbm,
          src_v, dst_v, srcx_v, dstx_v, rows_v, rows_h, hs_s, acc,
          sem, hsem):
    c = lax.axis_index("c")
    s = lax.axis_index("s")
    tid = c * NS + s
    sl = pl.ds(s * rows_per_tile, rows_per_tile)
    pltpu.sync_copy(hs_hbm.at[sl], hs_s.at[sl])
    pltpu.sync_copy(zeros_hbm.at[sl], acc.at[sl])
    pltpu.sync_copy(ei_hbm.at[0, pl.ds(tid * kb * CHUNK, kb * CHUNK)], src_v)
    pltpu.sync_copy(dst_hbm.at[pl.ds(tid * kb, kb)], dst_v)
    if rem:
      @pl.when(tid < rem)
      def _():
        pltpu.sync_copy(
            ei_hbm.at[0, pl.ds((NC * NS * kb + tid) * CHUNK, CHUNK)], srcx_v)
        pltpu.sync_copy(dst_hbm.at[NC * NS * kb + tid], dstx_v)
    plsc.subcore_barrier()

    def step(sv, dv):
      pltpu.async_copy(hs_s.at[sv], rows_v, sem).wait()
      pltpu.sync_copy(rows_v, acc.at[dv], add=True)

    def hbm_gather(j, b):
      return pltpu.make_async_copy(
          hs_hbm.at[src_v.at[pl.ds(j * CHUNK, CHUNK)]], rows_h[b], hsem[b])

    def group_body(jj, carry):
      j0 = 4 * jj
      hbm_gather(j0 + 1, 0).start()
      hbm_gather(j0 + 2, 1).start()
      step(src_v.at[pl.ds(j0 * CHUNK, CHUNK)], dst_v.at[j0])
      hbm_gather(j0 + 1, 0).wait()
      pltpu.sync_copy(rows_h[0], acc.at[dst_v.at[j0 + 1]], add=True)
      hbm_gather(j0 + 3, 0).start()
      hbm_gather(j0 + 2, 1).wait()
      pltpu.sync_copy(rows_h[1], acc.at[dst_v.at[j0 + 2]], add=True)
      hbm_gather(j0 + 3, 0).wait()
      pltpu.sync_copy(rows_h[0], acc.at[dst_v.at[j0 + 3]], add=True)
      return carry

    lax.fori_loop(0, kb // 4, group_body, 0)
    for j in range(kb - kb % 4, kb):
      step(src_v.at[pl.ds(j * CHUNK, CHUNK)], dst_v.at[j])
    if rem:
      @pl.when(tid < rem)
      def _():
        step(srcx_v, dstx_v)
    plsc.subcore_barrier()
    pltpu.sync_copy(acc.at[sl], out_hbm.at[c, sl])

  return agg


def _make_deg(n_pad, kb, rem, w=DEGW):
  mesh = plsc.VectorSubcoreMesh(core_axis_name="c", subcore_axis_name="s")
  rows_per_tile = n_pad // NS

  @functools.partial(
      pl.kernel,
      mesh=mesh,
      out_type=jax.ShapeDtypeStruct((NC, n_pad, w), jnp.float32),
      compiler_params=pltpu.CompilerParams(use_tc_tiling_on_sc=False),
      scratch_types=[
          pltpu.VMEM((kb, CHUNK), jnp.int32),
          pltpu.VMEM((CHUNK,), jnp.int32),
          pltpu.VMEM((CHUNK, w), jnp.float32),
          pltpu.VMEM_SHARED((n_pad, w), jnp.float32),
      ],
  )
  def deg(dst_hbm, ones_hbm, zeros_hbm, out_hbm, dst_v, dstx_v, ones_v, acc):
    c = lax.axis_index("c")
    s = lax.axis_index("s")
    tid = c * NS + s
    pltpu.sync_copy(ones_hbm, ones_v)
    sl = pl.ds(s * rows_per_tile, rows_per_tile)
    pltpu.sync_copy(zeros_hbm.at[pl.ds(0, rows_per_tile)], acc.at[sl])
    pltpu.sync_copy(dst_hbm.at[pl.ds(tid * kb, kb)], dst_v)
    if rem:
      @pl.when(tid < rem)
      def _():
        pltpu.sync_copy(dst_hbm.at[NC * NS * kb + tid], dstx_v)
    plsc.subcore_barrier()

    def chunk_body(j, carry):
      pltpu.sync_copy(ones_v, acc.at[dst_v.at[j]], add=True)
      return carry

    lax.fori_loop(0, kb, chunk_body, 0)
    if rem:
      @pl.when(tid < rem)
      def _():
        pltpu.sync_copy(ones_v, acc.at[dstx_v], add=True)
    plsc.subcore_barrier()
    pltpu.sync_copy(acc.at[sl], out_hbm.at[c, sl])

  return deg


def _tc_first(x, w1, deg2, block):
  n = x.shape[0]
  d_in, d_hid = w1.shape
  degw = deg2.shape[2]

  def body(x_ref, da_ref, db_ref, w_ref, hs_ref, dis_ref):
    deg = da_ref[0, :, :1] + db_ref[0, :, :1] + 1.0
    dis = lax.rsqrt(deg)
    h = jnp.dot(x_ref[...], w_ref[...], preferred_element_type=jnp.float32)
    hs_ref[...] = h * dis
    dis_ref[...] = dis

  grid = (n // block,)
  return pl.pallas_call(
      body,
      grid=grid,
      in_specs=[
          pl.BlockSpec((block, d_in), lambda i: (i, 0)),
          pl.BlockSpec((1, block, degw), lambda i: (0, i, 0)),
          pl.BlockSpec((1, block, degw), lambda i: (1, i, 0)),
          pl.BlockSpec((d_in, d_hid), lambda i: (0, 0)),
      ],
      out_specs=[
          pl.BlockSpec((block, d_hid), lambda i: (i, 0)),
          pl.BlockSpec((block, 1), lambda i: (i, 0)),
      ],
      out_shape=[
          jax.ShapeDtypeStruct((n, d_hid), jnp.float32),
          jax.ShapeDtypeStruct((n, 1), jnp.float32),
      ],
  )(x, deg2, deg2, w1)


def _tc_mid(p2, hs1, dis, b1, w2, block):
  n, d_hid = hs1.shape
  d_out = w2.shape[1]

  def body(pa_ref, pb_ref, hs_ref, dis_ref, b_ref, w_ref, hs2_ref):
    dis = dis_ref[...]
    z = dis * (pa_ref[0] + pb_ref[0] + hs_ref[...]) + b_ref[...]
    z = jnp.maximum(z, 0.0)
    h2 = jnp.dot(z, w_ref[...], preferred_element_type=jnp.float32)
    hs2_ref[...] = h2 * dis

  grid = (n // block,)
  return pl.pallas_call(
      body,
      grid=grid,
      in_specs=[
          pl.BlockSpec((1, block, d_hid), lambda i: (0, i, 0)),
          pl.BlockSpec((1, block, d_hid), lambda i: (1, i, 0)),
          pl.BlockSpec((block, d_hid), lambda i: (i, 0)),
          pl.BlockSpec((block, 1), lambda i: (i, 0)),
          pl.BlockSpec((1, d_hid), lambda i: (0, 0)),
          pl.BlockSpec((d_hid, d_out), lambda i: (0, 0)),
      ],
      out_specs=pl.BlockSpec((block, d_out), lambda i: (i, 0)),
      out_shape=jax.ShapeDtypeStruct((n, d_out), jnp.float32),
  )(p2, p2, hs1, dis, b1, w2)


def _tc_last(q2, hs2, dis, b2, block):
  n, d_out = hs2.shape

  def body(qa_ref, qb_ref, hs_ref, dis_ref, b_ref, o_ref):
    dis = dis_ref[...]
    o_ref[...] = dis * (qa_ref[0] + qb_ref[0] + hs_ref[...]) + b_ref[...]

  grid = (n // block,)
  return pl.pallas_call(
      body,
      grid=grid,
      in_specs=[
          pl.BlockSpec((1, block, d_out), lambda i: (0, i, 0)),
          pl.BlockSpec((1, block, d_out), lambda i: (1, i, 0)),
          pl.BlockSpec((block, d_out), lambda i: (i, 0)),
          pl.BlockSpec((block, 1), lambda i: (i, 0)),
          pl.BlockSpec((1, d_out), lambda i: (0, 0)),
      ],
      out_specs=pl.BlockSpec((block, d_out), lambda i: (i, 0)),
      out_shape=jax.ShapeDtypeStruct((n, d_out), jnp.float32),
  )(q2, q2, hs2, dis, b2)


def kernel(x, edge_index, W1, b1, W2, b2):
  n, d_in = x.shape
  d_hid = W1.shape[1]
  d_out = W2.shape[1]
  e = edge_index.shape[1]

  ei = edge_index.astype(jnp.int32)
  if e % CHUNK:
    pad = CHUNK - e % CHUNK
    ei = jnp.concatenate(
        [ei, jnp.tile(jnp.array([[0], [n]], jnp.int32), (1, pad))], axis=1)
    e += pad
  ktot = e // CHUNK
  kb = ktot // (NC * NS)
  rem = ktot - NC * NS * kb
  dst_arr = ei[1].reshape(ktot, CHUNK)

  n_pad = -(-(n + 1) // (NS * 8)) * (NS * 8)

  zeros_big = jnp.zeros((n_pad, max(d_hid, d_out)), jnp.float32)
  ones_col = jnp.ones((CHUNK, DEGW), jnp.float32)
  zeros_col = jnp.zeros((n_pad, DEGW), jnp.float32)
  x_p = jnp.pad(x, ((0, n_pad - n), (0, 0)))

  deg2 = _make_deg(n_pad, kb, rem)(dst_arr, ones_col, zeros_col)

  block = n_pad // 8
  hs1, dis = _tc_first(x_p, W1, deg2, block)

  agg1 = _make_agg(n_pad, d_hid, kb, rem)(
      hs1, ei, dst_arr, zeros_big[:, :d_hid])
  hs2 = _tc_mid(agg1, hs1, dis, b1.reshape(1, d_hid), W2, block)

  agg2 = _make_agg(n_pad, d_out, kb, rem)(
      hs2, ei, dst_arr, zeros_big[:, :d_out])
  out = _tc_last(agg2, hs2, dis, b2.reshape(1, d_out), block)
  return out[:n]


# speedup vs baseline: 1.0247x; 1.0247x over previous
import functools

import jax
import jax.numpy as jnp
from jax import lax
from jax.experimental import pallas as pl
from jax.experimental.pallas import tpu as pltpu
from jax.experimental.pallas import tpu_sc as plsc

NC = 2
NS = 16
CHUNK = 128
DEGW = 8


def _make_agg(n_pad, d, kb, rem):
  mesh = plsc.VectorSubcoreMesh(core_axis_name="c", subcore_axis_name="s")
  rows_per_tile = n_pad // NS

  @functools.partial(
      pl.kernel,
      mesh=mesh,
      out_type=jax.ShapeDtypeStruct((NC, n_pad, d), jnp.float32),
      compiler_params=pltpu.CompilerParams(use_tc_tiling_on_sc=False),
      scratch_types=[
          pltpu.VMEM((kb * CHUNK,), jnp.int32),
          pltpu.VMEM((kb, CHUNK), jnp.int32),
          pltpu.VMEM((CHUNK,), jnp.int32),
          pltpu.VMEM((CHUNK,), jnp.int32),
          pltpu.VMEM((CHUNK, d), jnp.float32),
          pltpu.VMEM((CHUNK, d), jnp.float32),
          pltpu.VMEM((CHUNK, d), jnp.float32),
          pltpu.VMEM_SHARED((n_pad, d), jnp.float32),
          pltpu.VMEM_SHARED((n_pad, d), jnp.float32),
          pltpu.SemaphoreType.DMA,
          pltpu.SemaphoreType.DMA,
          pltpu.SemaphoreType.DMA,
      ],
  )
  def agg(hs_hbm, ei_hbm, dst_hbm, zeros_hbm, out_hbm,
          src_v, dst_v, srcx_v, dstx_v, rows_v, rows_h, rows_h2, hs_s, acc,
          sem, hsem, hsem2):
    c = lax.axis_index("c")
    s = lax.axis_index("s")
    tid = c * NS + s
    sl = pl.ds(s * rows_per_tile, rows_per_tile)
    pltpu.sync_copy(hs_hbm.at[sl], hs_s.at[sl])
    pltpu.sync_copy(zeros_hbm.at[sl], acc.at[sl])
    pltpu.sync_copy(ei_hbm.at[0, pl.ds(tid * kb * CHUNK, kb * CHUNK)], src_v)
    pltpu.sync_copy(dst_hbm.at[pl.ds(tid * kb, kb)], dst_v)
    if rem:
      @pl.when(tid < rem)
      def _():
        pltpu.sync_copy(
            ei_hbm.at[0, pl.ds((NC * NS * kb + tid) * CHUNK, CHUNK)], srcx_v)
        pltpu.sync_copy(dst_hbm.at[NC * NS * kb + tid], dstx_v)
    plsc.subcore_barrier()

    def step(sv, dv):
      pltpu.async_copy(hs_s.at[sv], rows_v, sem).wait()
      pltpu.sync_copy(rows_v, acc.at[dv], add=True)

    def triple_body(jj, carry):
      j0 = 3 * jj
      h1 = pltpu.make_async_copy(
          hs_hbm.at[src_v.at[pl.ds((j0 + 1) * CHUNK, CHUNK)]], rows_h, hsem)
      h2 = pltpu.make_async_copy(
          hs_hbm.at[src_v.at[pl.ds((j0 + 2) * CHUNK, CHUNK)]], rows_h2, hsem2)
      h1.start()
      h2.start()
      step(src_v.at[pl.ds(j0 * CHUNK, CHUNK)], dst_v.at[j0])
      h1.wait()
      pltpu.sync_copy(rows_h, acc.at[dst_v.at[j0 + 1]], add=True)
      h2.wait()
      pltpu.sync_copy(rows_h2, acc.at[dst_v.at[j0 + 2]], add=True)
      return carry

    lax.fori_loop(0, kb // 3, triple_body, 0)
    for j in range(kb - kb % 3, kb):
      step(src_v.at[pl.ds(j * CHUNK, CHUNK)], dst_v.at[j])
    if rem:
      @pl.when(tid < rem)
      def _():
        step(srcx_v, dstx_v)
    plsc.subcore_barrier()
    pltpu.sync_copy(acc.at[sl], out_hbm.at[c, sl])

  return agg


def _make_deg(n_pad, kb, rem, w=DEGW):
  mesh = plsc.VectorSubcoreMesh(core_axis_name="c", subcore_axis_name="s")
  rows_per_tile = n_pad // NS

  @functools.partial(
      pl.kernel,
      mesh=mesh,
      out_type=jax.ShapeDtypeStruct((NC, n_pad, w), jnp.float32),
      compiler_params=pltpu.CompilerParams(use_tc_tiling_on_sc=False),
      scratch_types=[
          pltpu.VMEM((kb, CHUNK), jnp.int32),
          pltpu.VMEM((CHUNK,), jnp.int32),
          pltpu.VMEM((CHUNK, w), jnp.float32),
          pltpu.VMEM_SHARED((n_pad, w), jnp.float32),
      ],
  )
  def deg(dst_hbm, ones_hbm, zeros_hbm, out_hbm, dst_v, dstx_v, ones_v, acc):
    c = lax.axis_index("c")
    s = lax.axis_index("s")
    tid = c * NS + s
    pltpu.sync_copy(ones_hbm, ones_v)
    sl = pl.ds(s * rows_per_tile, rows_per_tile)
    pltpu.sync_copy(zeros_hbm.at[pl.ds(0, rows_per_tile)], acc.at[sl])
    pltpu.sync_copy(dst_hbm.at[pl.ds(tid * kb, kb)], dst_v)
    if rem:
      @pl.when(tid < rem)
      def _():
        pltpu.sync_copy(dst_hbm.at[NC * NS * kb + tid], dstx_v)
    plsc.subcore_barrier()

    def chunk_body(j, carry):
      pltpu.sync_copy(ones_v, acc.at[dst_v.at[j]], add=True)
      return carry

    lax.fori_loop(0, kb, chunk_body, 0)
    if rem:
      @pl.when(tid < rem)
      def _():
        pltpu.sync_copy(ones_v, acc.at[dstx_v], add=True)
    plsc.subcore_barrier()
    pltpu.sync_copy(acc.at[sl], out_hbm.at[c, sl])

  return deg


def _tc_first(x, w1, deg2, block):
  n = x.shape[0]
  d_in, d_hid = w1.shape
  degw = deg2.shape[2]

  def body(x_ref, da_ref, db_ref, w_ref, hs_ref, dis_ref):
    deg = da_ref[0, :, :1] + db_ref[0, :, :1] + 1.0
    dis = lax.rsqrt(deg)
    h = jnp.dot(x_ref[...], w_ref[...], preferred_element_type=jnp.float32)
    hs_ref[...] = h * dis
    dis_ref[...] = dis

  grid = (n // block,)
  return pl.pallas_call(
      body,
      grid=grid,
      in_specs=[
          pl.BlockSpec((block, d_in), lambda i: (i, 0)),
          pl.BlockSpec((1, block, degw), lambda i: (0, i, 0)),
          pl.BlockSpec((1, block, degw), lambda i: (1, i, 0)),
          pl.BlockSpec((d_in, d_hid), lambda i: (0, 0)),
      ],
      out_specs=[
          pl.BlockSpec((block, d_hid), lambda i: (i, 0)),
          pl.BlockSpec((block, 1), lambda i: (i, 0)),
      ],
      out_shape=[
          jax.ShapeDtypeStruct((n, d_hid), jnp.float32),
          jax.ShapeDtypeStruct((n, 1), jnp.float32),
      ],
  )(x, deg2, deg2, w1)


def _tc_mid(p2, hs1, dis, b1, w2, block):
  n, d_hid = hs1.shape
  d_out = w2.shape[1]

  def body(pa_ref, pb_ref, hs_ref, dis_ref, b_ref, w_ref, hs2_ref):
    dis = dis_ref[...]
    z = dis * (pa_ref[0] + pb_ref[0] + hs_ref[...]) + b_ref[...]
    z = jnp.maximum(z, 0.0)
    h2 = jnp.dot(z, w_ref[...], preferred_element_type=jnp.float32)
    hs2_ref[...] = h2 * dis

  grid = (n // block,)
  return pl.pallas_call(
      body,
      grid=grid,
      in_specs=[
          pl.BlockSpec((1, block, d_hid), lambda i: (0, i, 0)),
          pl.BlockSpec((1, block, d_hid), lambda i: (1, i, 0)),
          pl.BlockSpec((block, d_hid), lambda i: (i, 0)),
          pl.BlockSpec((block, 1), lambda i: (i, 0)),
          pl.BlockSpec((1, d_hid), lambda i: (0, 0)),
          pl.BlockSpec((d_hid, d_out), lambda i: (0, 0)),
      ],
      out_specs=pl.BlockSpec((block, d_out), lambda i: (i, 0)),
      out_shape=jax.ShapeDtypeStruct((n, d_out), jnp.float32),
  )(p2, p2, hs1, dis, b1, w2)


def _tc_last(q2, hs2, dis, b2, block):
  n, d_out = hs2.shape

  def body(qa_ref, qb_ref, hs_ref, dis_ref, b_ref, o_ref):
    dis = dis_ref[...]
    o_ref[...] = dis * (qa_ref[0] + qb_ref[0] + hs_ref[...]) + b_ref[...]

  grid = (n // block,)
  return pl.pallas_call(
      body,
      grid=grid,
      in_specs=[
          pl.BlockSpec((1, block, d_out), lambda i: (0, i, 0)),
          pl.BlockSpec((1, block, d_out), lambda i: (1, i, 0)),
          pl.BlockSpec((block, d_out), lambda i: (i, 0)),
          pl.BlockSpec((block, 1), lambda i: (i, 0)),
          pl.BlockSpec((1, d_out), lambda i: (0, 0)),
      ],
      out_specs=pl.BlockSpec((block, d_out), lambda i: (i, 0)),
      out_shape=jax.ShapeDtypeStruct((n, d_out), jnp.float32),
  )(q2, q2, hs2, dis, b2)


def kernel(x, edge_index, W1, b1, W2, b2):
  n, d_in = x.shape
  d_hid = W1.shape[1]
  d_out = W2.shape[1]
  e = edge_index.shape[1]

  ei = edge_index.astype(jnp.int32)
  if e % CHUNK:
    pad = CHUNK - e % CHUNK
    ei = jnp.concatenate(
        [ei, jnp.tile(jnp.array([[0], [n]], jnp.int32), (1, pad))], axis=1)
    e += pad
  ktot = e // CHUNK
  kb = ktot // (NC * NS)
  rem = ktot - NC * NS * kb
  dst_arr = ei[1].reshape(ktot, CHUNK)

  n_pad = -(-(n + 1) // (NS * 8)) * (NS * 8)

  zeros_big = jnp.zeros((n_pad, max(d_hid, d_out)), jnp.float32)
  ones_col = jnp.ones((CHUNK, DEGW), jnp.float32)
  zeros_col = jnp.zeros((n_pad, DEGW), jnp.float32)
  x_p = jnp.pad(x, ((0, n_pad - n), (0, 0)))

  deg2 = _make_deg(n_pad, kb, rem)(dst_arr, ones_col, zeros_col)

  block = n_pad // 8
  hs1, dis = _tc_first(x_p, W1, deg2, block)

  agg1 = _make_agg(n_pad, d_hid, kb, rem)(
      hs1, ei, dst_arr, zeros_big[:, :d_hid])
  hs2 = _tc_mid(agg1, hs1, dis, b1.reshape(1, d_hid), W2, block)

  agg2 = _make_agg(n_pad, d_out, kb, rem)(
      hs2, ei, dst_arr, zeros_big[:, :d_out])
  out = _tc_last(agg2, hs2, dis, b2.reshape(1, d_out), block)
  return out[:n]
